# baseline (device time: 63490 ns/iter reference)
import jax
import jax.numpy as jnp
from jax import lax
from jax.experimental import pallas as pl
from jax.experimental.pallas import tpu as pltpu

N_DEV = 8
B, SQ, SKV = 2, 512, 512
HQ_PER = 8
DH = 64
DMODEL = 768
DHEADS = HQ_PER * DH
N_CHUNK = 8
ROWS = B * SQ
CHUNK_ROWS = ROWS // N_CHUNK


def kernel(x, Wq, K_ext, V_ext, Wo):
    me = lax.axis_index("i")
    Wq_loc = lax.dynamic_slice(Wq, (0, me * DHEADS), (DMODEL, DHEADS))
    Wo_loc = lax.dynamic_slice(Wo, (me * DHEADS, 0), (DHEADS, DMODEL))
    K_t = K_ext.transpose(0, 2, 1, 3)
    V_t = V_ext.transpose(0, 2, 1, 3)

    def body(x_ref, wq_ref, k_ref, v_ref, wo_ref, out_ref,
             acc_ref, rbuf0, rbuf1, rbuf2, q_ref, ctx_ref,
             send_sems, recv_sems):
        my = lax.axis_index("i")

        rowb = lax.broadcasted_iota(jnp.int32, (SQ, SKV), 0) // 64
        colb = lax.broadcasted_iota(jnp.int32, (SQ, SKV), 1) // 64
        mask = colb <= rowb

        for b in range(B):
            q_ref[...] = jnp.dot(
                x_ref[b], wq_ref[...], preferred_element_type=jnp.float32
            )
            for h in range(HQ_PER):
                qh = q_ref[:, h * DH:(h + 1) * DH]
                s = lax.dot_general(
                    qh, k_ref[b, h], (((1,), (1,)), ((), ())),
                    preferred_element_type=jnp.float32,
                ) * 0.125
                s = jnp.where(mask, s, -1e9)
                m = jnp.max(s, axis=-1, keepdims=True)
                e = jnp.exp(s - m)
                w = e / jnp.sum(e, axis=-1, keepdims=True)
                ctx_ref[b * SQ:(b + 1) * SQ, h * DH:(h + 1) * DH] = jnp.dot(
                    w, v_ref[b, h], preferred_element_type=jnp.float32
                )

        def proj_chunk(c):
            rows = ctx_ref[pl.ds(c * CHUNK_ROWS, CHUNK_ROWS), :]
            part = jnp.dot(rows, wo_ref[...], preferred_element_type=jnp.float32)
            acc_ref[pl.ds(c, 1)] = part.astype(jnp.bfloat16)[None]

        def make_exchange(step, partner, src_ref, dst_ref):
            return pltpu.make_async_remote_copy(
                src_ref=src_ref,
                dst_ref=dst_ref,
                send_sem=send_sems.at[step],
                recv_sem=recv_sems.at[step],
                device_id=(partner,),
                device_id_type=pl.DeviceIdType.MESH,
            )

        def accum(c, src):
            acc_ref[pl.ds(c, 1)] = (
                acc_ref[pl.ds(c, 1)].astype(jnp.float32)
                + src.astype(jnp.float32)
            ).astype(jnp.bfloat16)

        p0 = my ^ 4
        send_base = p0 & 4
        keep_base = my & 4
        for j in range(4):
            proj_chunk(send_base + j)

        barrier = pltpu.get_barrier_semaphore()
        for d in (1, 2, 4):
            pl.semaphore_signal(
                barrier, inc=1,
                device_id=(my ^ d,), device_id_type=pl.DeviceIdType.MESH,
            )
        pl.semaphore_wait(barrier, 3)

        rdma0 = make_exchange(0, p0, acc_ref.at[pl.ds(send_base, 4)], rbuf0)
        rdma0.start()
        for j in range(4):
            proj_chunk(keep_base + j)
        rdma0.wait()
        for j in range(4):
            accum(keep_base + j, rbuf0[pl.ds(j, 1)])

        p1 = my ^ 2
        rdma1 = make_exchange(
            1, p1, acc_ref.at[pl.ds((my & 4) | (p1 & 2), 2)], rbuf1
        )
        rdma1.start()
        rdma1.wait()
        kb = my & 6
        for j in range(2):
            accum(kb + j, rbuf1[pl.ds(j, 1)])

        p2 = my ^ 1
        rdma2 = make_exchange(2, p2, acc_ref.at[pl.ds(p2, 1)], rbuf2)
        rdma2.start()
        rdma2.wait()
        accum(my, rbuf2[pl.ds(0, 1)])

        rdma3 = make_exchange(
            3, p2, acc_ref.at[pl.ds(my, 1)], acc_ref.at[pl.ds(my, 1)]
        )
        rdma3.start()
        rdma3.wait()
        rdma4 = make_exchange(
            4, p1, acc_ref.at[pl.ds(my & 6, 2)], acc_ref.at[pl.ds(my & 6, 2)]
        )
        rdma4.start()
        rdma4.wait()
        rdma5 = make_exchange(
            5, p0, acc_ref.at[pl.ds(my & 4, 4)], acc_ref.at[pl.ds(my & 4, 4)]
        )
        rdma5.start()
        out_ref[pl.ds(keep_base * CHUNK_ROWS, 4 * CHUNK_ROWS), :] = (
            acc_ref[pl.ds(keep_base, 4)]
            .astype(jnp.float32)
            .reshape(4 * CHUNK_ROWS, DMODEL)
        )
        rdma5.wait()
        out_ref[pl.ds(send_base * CHUNK_ROWS, 4 * CHUNK_ROWS), :] = (
            acc_ref[pl.ds(send_base, 4)]
            .astype(jnp.float32)
            .reshape(4 * CHUNK_ROWS, DMODEL)
        )

    out_flat = pl.pallas_call(
        body,
        out_shape=jax.ShapeDtypeStruct((ROWS, DMODEL), jnp.float32),
        in_specs=[pl.BlockSpec(memory_space=pltpu.VMEM)] * 5,
        out_specs=pl.BlockSpec(memory_space=pltpu.VMEM),
        scratch_shapes=[
            pltpu.VMEM((N_CHUNK, CHUNK_ROWS, DMODEL), jnp.bfloat16),
            pltpu.VMEM((4, CHUNK_ROWS, DMODEL), jnp.bfloat16),
            pltpu.VMEM((2, CHUNK_ROWS, DMODEL), jnp.bfloat16),
            pltpu.VMEM((1, CHUNK_ROWS, DMODEL), jnp.bfloat16),
            pltpu.VMEM((SQ, DHEADS), jnp.float32),
            pltpu.VMEM((ROWS, DHEADS), jnp.float32),
            pltpu.SemaphoreType.DMA((6,)),
            pltpu.SemaphoreType.DMA((6,)),
        ],
        compiler_params=pltpu.CompilerParams(collective_id=0),
    )(x, Wq_loc, K_t, V_t, Wo_loc)
    return out_flat.reshape(B, SQ, DMODEL)


# device time: 61305 ns/iter; 1.0356x vs baseline; 1.0356x over previous
import jax
import jax.numpy as jnp
from jax import lax
from jax.experimental import pallas as pl
from jax.experimental.pallas import tpu as pltpu

N_DEV = 8
B, SQ, SKV = 2, 512, 512
HQ_PER = 8
DH = 64
DMODEL = 768
DHEADS = HQ_PER * DH
N_CHUNK = 8
ROWS = B * SQ
CHUNK_ROWS = ROWS // N_CHUNK


def kernel(x, Wq, K_ext, V_ext, Wo):
    me = lax.axis_index("i")
    bf16 = jnp.bfloat16
    Wq_loc = lax.dynamic_slice(Wq, (0, me * DHEADS), (DMODEL, DHEADS)).astype(bf16)
    Wo_loc = lax.dynamic_slice(Wo, (me * DHEADS, 0), (DHEADS, DMODEL)).astype(bf16)
    K_t = K_ext.transpose(0, 2, 1, 3).astype(bf16)
    V_t = V_ext.transpose(0, 2, 1, 3).astype(bf16)
    x16 = x.astype(bf16)

    def body(x_ref, wq_ref, k_ref, v_ref, wo_ref, out_ref,
             acc_ref, rbuf0, rbuf1, rbuf2, q_ref, ctx_ref,
             send_sems, recv_sems):
        my = lax.axis_index("i")

        rowb = lax.broadcasted_iota(jnp.int32, (SQ, SKV), 0) // 64
        colb = lax.broadcasted_iota(jnp.int32, (SQ, SKV), 1) // 64
        mask = colb <= rowb

        for b in range(B):
            q_ref[...] = jnp.dot(
                x_ref[b], wq_ref[...], preferred_element_type=jnp.float32
            ).astype(jnp.bfloat16)
            for h in range(HQ_PER):
                qh = q_ref[:, h * DH:(h + 1) * DH]
                s = lax.dot_general(
                    qh, k_ref[b, h], (((1,), (1,)), ((), ())),
                    preferred_element_type=jnp.float32,
                ) * 0.125
                s = jnp.where(mask, s, -1e9)
                m = jnp.max(s, axis=-1, keepdims=True)
                e = jnp.exp(s - m)
                w = (e / jnp.sum(e, axis=-1, keepdims=True)).astype(jnp.bfloat16)
                ctx_ref[b * SQ:(b + 1) * SQ, h * DH:(h + 1) * DH] = jnp.dot(
                    w, v_ref[b, h], preferred_element_type=jnp.float32
                ).astype(jnp.bfloat16)

        def proj_chunk(c):
            rows = ctx_ref[pl.ds(c * CHUNK_ROWS, CHUNK_ROWS), :]
            part = jnp.dot(rows, wo_ref[...], preferred_element_type=jnp.float32)
            acc_ref[pl.ds(c, 1)] = part.astype(jnp.bfloat16)[None]

        def make_exchange(step, partner, src_ref, dst_ref):
            return pltpu.make_async_remote_copy(
                src_ref=src_ref,
                dst_ref=dst_ref,
                send_sem=send_sems.at[step],
                recv_sem=recv_sems.at[step],
                device_id=(partner,),
                device_id_type=pl.DeviceIdType.MESH,
            )

        def accum(c, src):
            acc_ref[pl.ds(c, 1)] = (
                acc_ref[pl.ds(c, 1)].astype(jnp.float32)
                + src.astype(jnp.float32)
            ).astype(jnp.bfloat16)

        p0 = my ^ 4
        send_base = p0 & 4
        keep_base = my & 4
        for j in range(4):
            proj_chunk(send_base + j)

        barrier = pltpu.get_barrier_semaphore()
        for d in (1, 2, 4):
            pl.semaphore_signal(
                barrier, inc=1,
                device_id=(my ^ d,), device_id_type=pl.DeviceIdType.MESH,
            )
        pl.semaphore_wait(barrier, 3)

        rdma0 = make_exchange(0, p0, acc_ref.at[pl.ds(send_base, 4)], rbuf0)
        rdma0.start()
        for j in range(4):
            proj_chunk(keep_base + j)
        rdma0.wait()
        for j in range(4):
            accum(keep_base + j, rbuf0[pl.ds(j, 1)])

        p1 = my ^ 2
        rdma1 = make_exchange(
            1, p1, acc_ref.at[pl.ds((my & 4) | (p1 & 2), 2)], rbuf1
        )
        rdma1.start()
        rdma1.wait()
        kb = my & 6
        for j in range(2):
            accum(kb + j, rbuf1[pl.ds(j, 1)])

        p2 = my ^ 1
        rdma2 = make_exchange(2, p2, acc_ref.at[pl.ds(p2, 1)], rbuf2)
        rdma2.start()
        rdma2.wait()
        accum(my, rbuf2[pl.ds(0, 1)])

        rdma3 = make_exchange(
            3, p2, acc_ref.at[pl.ds(my, 1)], acc_ref.at[pl.ds(my, 1)]
        )
        rdma3.start()
        rdma3.wait()
        rdma4 = make_exchange(
            4, p1, acc_ref.at[pl.ds(my & 6, 2)], acc_ref.at[pl.ds(my & 6, 2)]
        )
        rdma4.start()
        rdma4.wait()
        rdma5 = make_exchange(
            5, p0, acc_ref.at[pl.ds(my & 4, 4)], acc_ref.at[pl.ds(my & 4, 4)]
        )
        rdma5.start()
        out_ref[pl.ds(keep_base * CHUNK_ROWS, 4 * CHUNK_ROWS), :] = (
            acc_ref[pl.ds(keep_base, 4)]
            .astype(jnp.float32)
            .reshape(4 * CHUNK_ROWS, DMODEL)
        )
        rdma5.wait()
        out_ref[pl.ds(send_base * CHUNK_ROWS, 4 * CHUNK_ROWS), :] = (
            acc_ref[pl.ds(send_base, 4)]
            .astype(jnp.float32)
            .reshape(4 * CHUNK_ROWS, DMODEL)
        )

    out_flat = pl.pallas_call(
        body,
        out_shape=jax.ShapeDtypeStruct((ROWS, DMODEL), jnp.float32),
        in_specs=[pl.BlockSpec(memory_space=pltpu.VMEM)] * 5,
        out_specs=pl.BlockSpec(memory_space=pltpu.VMEM),
        scratch_shapes=[
            pltpu.VMEM((N_CHUNK, CHUNK_ROWS, DMODEL), jnp.bfloat16),
            pltpu.VMEM((4, CHUNK_ROWS, DMODEL), jnp.bfloat16),
            pltpu.VMEM((2, CHUNK_ROWS, DMODEL), jnp.bfloat16),
            pltpu.VMEM((1, CHUNK_ROWS, DMODEL), jnp.bfloat16),
            pltpu.VMEM((SQ, DHEADS), jnp.bfloat16),
            pltpu.VMEM((ROWS, DHEADS), jnp.bfloat16),
            pltpu.SemaphoreType.DMA((6,)),
            pltpu.SemaphoreType.DMA((6,)),
        ],
        compiler_params=pltpu.CompilerParams(collective_id=0),
    )(x16, Wq_loc, K_t, V_t, Wo_loc)
    return out_flat.reshape(B, SQ, DMODEL)


# device time: 47572 ns/iter; 1.3346x vs baseline; 1.2887x over previous
import jax
import jax.numpy as jnp
from jax import lax
from jax.experimental import pallas as pl
from jax.experimental.pallas import tpu as pltpu

N_DEV = 8
B, SQ, SKV = 2, 512, 512
HQ_PER = 8
DH = 64
DMODEL = 768
DHEADS = HQ_PER * DH
N_CHUNK = 8
ROWS = B * SQ
CHUNK_ROWS = ROWS // N_CHUNK
SEG = DMODEL // 2


def kernel(x, Wq, K_ext, V_ext, Wo):
    me = lax.axis_index("i")
    bf16 = jnp.bfloat16
    Wq_loc = lax.dynamic_slice(Wq, (0, me * DHEADS), (DMODEL, DHEADS)).astype(bf16)
    Wo_loc = lax.dynamic_slice(Wo, (me * DHEADS, 0), (DHEADS, DMODEL)).astype(bf16)
    K_t = K_ext.transpose(0, 2, 1, 3).astype(bf16)
    V_t = V_ext.transpose(0, 2, 1, 3).astype(bf16)
    x16 = x.astype(bf16)

    def body(x_ref, wq_ref, k_ref, v_ref, wo_ref, out_ref,
             accA, accB, rbA0, rbA1, rbA2, rbB0, rbB1, rbB2,
             q_ref, ctx_ref, send_sems, recv_sems):
        my = lax.axis_index("i")

        rowb = lax.broadcasted_iota(jnp.int32, (SQ, SKV), 0) // 64
        colb = lax.broadcasted_iota(jnp.int32, (SQ, SKV), 1) // 64
        mask = colb <= rowb

        for b in range(B):
            q_ref[...] = jnp.dot(
                x_ref[b], wq_ref[...], preferred_element_type=jnp.float32
            ).astype(jnp.bfloat16)
            for h in range(HQ_PER):
                qh = q_ref[:, h * DH:(h + 1) * DH]
                s = lax.dot_general(
                    qh, k_ref[b, h], (((1,), (1,)), ((), ())),
                    preferred_element_type=jnp.float32,
                ) * 0.125
                s = jnp.where(mask, s, -1e9)
                m = jnp.max(s, axis=-1, keepdims=True)
                e = jnp.exp(s - m)
                w = (e / jnp.sum(e, axis=-1, keepdims=True)).astype(jnp.bfloat16)
                ctx_ref[b * SQ:(b + 1) * SQ, h * DH:(h + 1) * DH] = jnp.dot(
                    w, v_ref[b, h], preferred_element_type=jnp.float32
                ).astype(jnp.bfloat16)

        def slotB(c):
            return ((c & 1) << 2) | (((c >> 2) & 1) << 1) | ((c >> 1) & 1)

        def chunkB(s):
            return ((s >> 2) & 1) | (((s >> 1) & 1) << 2) | ((s & 1) << 1)

        def proj_chunk(c):
            rows = ctx_ref[pl.ds(c * CHUNK_ROWS, CHUNK_ROWS), :]
            part = jnp.dot(rows, wo_ref[...], preferred_element_type=jnp.float32)
            accA[pl.ds(c, 1)] = part[:, :SEG].astype(jnp.bfloat16)[None]
            accB[pl.ds(slotB(c), 1)] = part[:, SEG:].astype(jnp.bfloat16)[None]

        def make_exchange(step, partner, src_ref, dst_ref):
            return pltpu.make_async_remote_copy(
                src_ref=src_ref,
                dst_ref=dst_ref,
                send_sem=send_sems.at[step],
                recv_sem=recv_sems.at[step],
                device_id=(partner,),
                device_id_type=pl.DeviceIdType.MESH,
            )

        def accum(ref, s, src):
            ref[pl.ds(s, 1)] = (
                ref[pl.ds(s, 1)].astype(jnp.float32)
                + src.astype(jnp.float32)
            ).astype(jnp.bfloat16)

        pz = my ^ 4
        pd = my ^ 2
        px = my ^ 1

        for j in range(4):
            proj_chunk((pz & 4) + j)

        barrier = pltpu.get_barrier_semaphore()
        for p in (px, pd, pz):
            pl.semaphore_signal(
                barrier, inc=1,
                device_id=(p,), device_id_type=pl.DeviceIdType.MESH,
            )
        pl.semaphore_wait(barrier, 3)

        a0 = make_exchange(0, pz, accA.at[pl.ds(pz & 4, 4)], rbA0)
        a0.start()
        for j in range(4):
            proj_chunk((my & 4) + j)
        b0 = make_exchange(6, px, accB.at[pl.ds((px & 1) * 4, 4)], rbB0)
        b0.start()

        a0.wait()
        kA0 = my & 4
        kA1 = my & 6
        sA1 = (my & 4) | (pd & 2)
        accum(accA, sA1, rbA0[pl.ds(sA1 - kA0, 1)])
        accum(accA, sA1 + 1, rbA0[pl.ds(sA1 - kA0 + 1, 1)])
        a1 = make_exchange(1, pd, accA.at[pl.ds(sA1, 2)], rbA1)
        a1.start()
        accum(accA, kA1, rbA0[pl.ds(kA1 - kA0, 1)])
        accum(accA, kA1 + 1, rbA0[pl.ds(kA1 - kA0 + 1, 1)])

        b0.wait()
        gB0 = (my & 1) * 4
        sB1 = gB0 + 2 * ((pz >> 2) & 1)
        kB1 = gB0 + 2 * ((my >> 2) & 1)
        accum(accB, sB1, rbB0[pl.ds(sB1 - gB0, 1)])
        accum(accB, sB1 + 1, rbB0[pl.ds(sB1 - gB0 + 1, 1)])
        b1 = make_exchange(7, pz, accB.at[pl.ds(sB1, 2)], rbB1)
        b1.start()
        accum(accB, kB1, rbB0[pl.ds(kB1 - gB0, 1)])
        accum(accB, kB1 + 1, rbB0[pl.ds(kB1 - gB0 + 1, 1)])

        a1.wait()
        accum(accA, px, rbA1[pl.ds(px - kA1, 1)])
        a2 = make_exchange(2, px, accA.at[pl.ds(px, 1)], rbA2)
        a2.start()
        accum(accA, my, rbA1[pl.ds(my - kA1, 1)])

        b1.wait()
        sBmy = slotB(my)
        sBpd = slotB(pd)
        accum(accB, sBpd, rbB1[pl.ds(sBpd - kB1, 1)])
        b2 = make_exchange(8, pd, accB.at[pl.ds(sBpd, 1)], rbB2)
        b2.start()
        accum(accB, sBmy, rbB1[pl.ds(sBmy - kB1, 1)])

        a2.wait()
        accum(accA, my, rbA2[pl.ds(0, 1)])
        a3 = make_exchange(3, px, accA.at[pl.ds(my, 1)], accA.at[pl.ds(my, 1)])
        a3.start()

        b2.wait()
        accum(accB, sBmy, rbB2[pl.ds(0, 1)])
        b3 = make_exchange(
            9, pd, accB.at[pl.ds(sBmy, 1)], accB.at[pl.ds(sBmy, 1)]
        )
        b3.start()

        a3.wait()
        a4 = make_exchange(
            4, pd, accA.at[pl.ds(kA1, 2)], accA.at[pl.ds(kA1, 2)]
        )
        a4.start()
        b3.wait()
        b4 = make_exchange(
            10, pz, accB.at[pl.ds(kB1, 2)], accB.at[pl.ds(kB1, 2)]
        )
        b4.start()
        a4.wait()
        a5 = make_exchange(
            5, pz, accA.at[pl.ds(kA0, 4)], accA.at[pl.ds(kA0, 4)]
        )
        a5.start()
        b4.wait()
        b5 = make_exchange(
            11, px, accB.at[pl.ds(gB0, 4)], accB.at[pl.ds(gB0, 4)]
        )
        b5.start()

        out_ref[pl.ds(kA0 * CHUNK_ROWS, 4 * CHUNK_ROWS), :SEG] = (
            accA[pl.ds(kA0, 4)]
            .astype(jnp.float32)
            .reshape(4 * CHUNK_ROWS, SEG)
        )
        for j in range(4):
            c = chunkB(gB0 + j)
            out_ref[pl.ds(c * CHUNK_ROWS, CHUNK_ROWS), SEG:] = (
                accB[gB0 + j].astype(jnp.float32)
            )

        a5.wait()
        out_ref[pl.ds((pz & 4) * CHUNK_ROWS, 4 * CHUNK_ROWS), :SEG] = (
            accA[pl.ds(pz & 4, 4)]
            .astype(jnp.float32)
            .reshape(4 * CHUNK_ROWS, SEG)
        )
        b5.wait()
        gB0r = (px & 1) * 4
        for j in range(4):
            c = chunkB(gB0r + j)
            out_ref[pl.ds(c * CHUNK_ROWS, CHUNK_ROWS), SEG:] = (
                accB[gB0r + j].astype(jnp.float32)
            )

    out_flat = pl.pallas_call(
        body,
        out_shape=jax.ShapeDtypeStruct((ROWS, DMODEL), jnp.float32),
        in_specs=[pl.BlockSpec(memory_space=pltpu.VMEM)] * 5,
        out_specs=pl.BlockSpec(memory_space=pltpu.VMEM),
        scratch_shapes=[
            pltpu.VMEM((N_CHUNK, CHUNK_ROWS, SEG), jnp.bfloat16),
            pltpu.VMEM((N_CHUNK, CHUNK_ROWS, SEG), jnp.bfloat16),
            pltpu.VMEM((4, CHUNK_ROWS, SEG), jnp.bfloat16),
            pltpu.VMEM((2, CHUNK_ROWS, SEG), jnp.bfloat16),
            pltpu.VMEM((1, CHUNK_ROWS, SEG), jnp.bfloat16),
            pltpu.VMEM((4, CHUNK_ROWS, SEG), jnp.bfloat16),
            pltpu.VMEM((2, CHUNK_ROWS, SEG), jnp.bfloat16),
            pltpu.VMEM((1, CHUNK_ROWS, SEG), jnp.bfloat16),
            pltpu.VMEM((SQ, DHEADS), jnp.bfloat16),
            pltpu.VMEM((ROWS, DHEADS), jnp.bfloat16),
            pltpu.SemaphoreType.DMA((12,)),
            pltpu.SemaphoreType.DMA((12,)),
        ],
        compiler_params=pltpu.CompilerParams(collective_id=0),
    )(x16, Wq_loc, K_t, V_t, Wo_loc)
    return out_flat.reshape(B, SQ, DMODEL)


# device time: 46103 ns/iter; 1.3771x vs baseline; 1.0319x over previous
import jax
import jax.numpy as jnp
from jax import lax
from jax.experimental import pallas as pl
from jax.experimental.pallas import tpu as pltpu

N_DEV = 8
B, SQ, SKV = 2, 512, 512
HQ_PER = 8
DH = 64
DMODEL = 768
DHEADS = HQ_PER * DH
N_CHUNK = 8
ROWS = B * SQ
CHUNK_ROWS = ROWS // N_CHUNK
SEG = DMODEL // 2


def kernel(x, Wq, K_ext, V_ext, Wo):
    me = lax.axis_index("i")
    bf16 = jnp.bfloat16
    Wq_loc = lax.dynamic_slice(Wq, (0, me * DHEADS), (DMODEL, DHEADS)).astype(bf16)
    Wo_loc = lax.dynamic_slice(Wo, (me * DHEADS, 0), (DHEADS, DMODEL)).astype(bf16)
    K_t = K_ext.transpose(0, 2, 1, 3).astype(bf16)
    V_t = V_ext.transpose(0, 2, 1, 3).astype(bf16)
    x16 = x.astype(bf16)

    def body(x_ref, wq_ref, k_ref, v_ref, wo_ref, out_ref,
             accA, accB, rbA0, rbA1, rbA2, rbB0, rbB1, rbB2,
             q_ref, ctx_ref, send_sems, recv_sems):
        my = lax.axis_index("i")

        rowb = lax.broadcasted_iota(jnp.int32, (SQ, SKV), 0) // 64
        colb = lax.broadcasted_iota(jnp.int32, (SQ, SKV), 1) // 64
        mask01 = (colb <= rowb).astype(jnp.float32)

        for b in range(B):
            q_ref[...] = (
                jnp.dot(
                    x_ref[b], wq_ref[...], preferred_element_type=jnp.float32
                ) * 0.125
            ).astype(jnp.bfloat16)
            for h in range(HQ_PER):
                qh = q_ref[:, h * DH:(h + 1) * DH]
                s = lax.dot_general(
                    qh, k_ref[b, h], (((1,), (1,)), ((), ())),
                    preferred_element_type=jnp.float32,
                )
                e = jnp.exp(s) * mask01
                w = (e / jnp.sum(e, axis=-1, keepdims=True)).astype(jnp.bfloat16)
                ctx_ref[b * SQ:(b + 1) * SQ, h * DH:(h + 1) * DH] = jnp.dot(
                    w, v_ref[b, h], preferred_element_type=jnp.float32
                ).astype(jnp.bfloat16)

        def slotB(c):
            return ((c & 1) << 2) | (((c >> 2) & 1) << 1) | ((c >> 1) & 1)

        def chunkB(s):
            return ((s >> 2) & 1) | (((s >> 1) & 1) << 2) | ((s & 1) << 1)

        def proj_chunk(c):
            rows = ctx_ref[pl.ds(c * CHUNK_ROWS, CHUNK_ROWS), :]
            part = jnp.dot(rows, wo_ref[...], preferred_element_type=jnp.float32)
            accA[pl.ds(c, 1)] = part[:, :SEG].astype(jnp.bfloat16)[None]
            accB[pl.ds(slotB(c), 1)] = part[:, SEG:].astype(jnp.bfloat16)[None]

        def make_exchange(step, partner, src_ref, dst_ref):
            return pltpu.make_async_remote_copy(
                src_ref=src_ref,
                dst_ref=dst_ref,
                send_sem=send_sems.at[step],
                recv_sem=recv_sems.at[step],
                device_id=(partner,),
                device_id_type=pl.DeviceIdType.MESH,
            )

        def accum(ref, s, rb, off, n=1):
            ref[pl.ds(s, n)] = (
                ref[pl.ds(s, n)].astype(jnp.float32)
                + rb[pl.ds(off, n)].astype(jnp.float32)
            ).astype(jnp.bfloat16)

        pz = my ^ 4
        pd = my ^ 2
        px = my ^ 1

        for j in range(4):
            proj_chunk((pz & 4) + j)

        barrier = pltpu.get_barrier_semaphore()
        for p in (px, pd, pz):
            pl.semaphore_signal(
                barrier, inc=1,
                device_id=(p,), device_id_type=pl.DeviceIdType.MESH,
            )
        pl.semaphore_wait(barrier, 3)

        a0 = make_exchange(0, pz, accA.at[pl.ds(pz & 4, 4)], rbA0)
        a0.start()
        for j in range(4):
            proj_chunk((my & 4) + j)
        b0 = make_exchange(6, px, accB.at[pl.ds((px & 1) * 4, 4)], rbB0)
        b0.start()

        a0.wait()
        kA0 = my & 4
        kA1 = my & 6
        sA1 = (my & 4) | (pd & 2)
        accum(accA, sA1, rbA0, sA1 - kA0, 2)
        a1 = make_exchange(1, pd, accA.at[pl.ds(sA1, 2)], rbA1)
        a1.start()
        accum(accA, kA1, rbA0, kA1 - kA0, 2)

        b0.wait()
        gB0 = (my & 1) * 4
        sB1 = gB0 + 2 * ((pz >> 2) & 1)
        kB1 = gB0 + 2 * ((my >> 2) & 1)
        accum(accB, sB1, rbB0, sB1 - gB0, 2)
        b1 = make_exchange(7, pz, accB.at[pl.ds(sB1, 2)], rbB1)
        b1.start()
        accum(accB, kB1, rbB0, kB1 - gB0, 2)

        a1.wait()
        accum(accA, px, rbA1, px - kA1)
        a2 = make_exchange(2, px, accA.at[pl.ds(px, 1)], rbA2)
        a2.start()
        accum(accA, my, rbA1, my - kA1)

        b1.wait()
        sBmy = slotB(my)
        sBpd = slotB(pd)
        accum(accB, sBpd, rbB1, sBpd - kB1)
        b2 = make_exchange(8, pd, accB.at[pl.ds(sBpd, 1)], rbB2)
        b2.start()
        accum(accB, sBmy, rbB1, sBmy - kB1)

        a2.wait()
        accum(accA, my, rbA2, 0)
        a3 = make_exchange(3, px, accA.at[pl.ds(my, 1)], accA.at[pl.ds(my, 1)])
        a3.start()

        b2.wait()
        accum(accB, sBmy, rbB2, 0)
        b3 = make_exchange(
            9, pd, accB.at[pl.ds(sBmy, 1)], accB.at[pl.ds(sBmy, 1)]
        )
        b3.start()

        a3.wait()
        a4 = make_exchange(
            4, pd, accA.at[pl.ds(kA1, 2)], accA.at[pl.ds(kA1, 2)]
        )
        a4.start()
        b3.wait()
        b4 = make_exchange(
            10, pz, accB.at[pl.ds(kB1, 2)], accB.at[pl.ds(kB1, 2)]
        )
        b4.start()
        a4.wait()
        a5 = make_exchange(
            5, pz, accA.at[pl.ds(kA0, 4)], accA.at[pl.ds(kA0, 4)]
        )
        a5.start()
        b4.wait()
        b5 = make_exchange(
            11, px, accB.at[pl.ds(gB0, 4)], accB.at[pl.ds(gB0, 4)]
        )
        b5.start()

        out_ref[pl.ds(kA0 * CHUNK_ROWS, 4 * CHUNK_ROWS), :SEG] = (
            accA[pl.ds(kA0, 4)]
            .astype(jnp.float32)
            .reshape(4 * CHUNK_ROWS, SEG)
        )
        for j in range(4):
            c = chunkB(gB0 + j)
            out_ref[pl.ds(c * CHUNK_ROWS, CHUNK_ROWS), SEG:] = (
                accB[gB0 + j].astype(jnp.float32)
            )

        a5.wait()
        out_ref[pl.ds((pz & 4) * CHUNK_ROWS, 4 * CHUNK_ROWS), :SEG] = (
            accA[pl.ds(pz & 4, 4)]
            .astype(jnp.float32)
            .reshape(4 * CHUNK_ROWS, SEG)
        )
        b5.wait()
        gB0r = (px & 1) * 4
        for j in range(4):
            c = chunkB(gB0r + j)
            out_ref[pl.ds(c * CHUNK_ROWS, CHUNK_ROWS), SEG:] = (
                accB[gB0r + j].astype(jnp.float32)
            )

    out_flat = pl.pallas_call(
        body,
        out_shape=jax.ShapeDtypeStruct((ROWS, DMODEL), jnp.float32),
        in_specs=[pl.BlockSpec(memory_space=pltpu.VMEM)] * 5,
        out_specs=pl.BlockSpec(memory_space=pltpu.VMEM),
        scratch_shapes=[
            pltpu.VMEM((N_CHUNK, CHUNK_ROWS, SEG), jnp.bfloat16),
            pltpu.VMEM((N_CHUNK, CHUNK_ROWS, SEG), jnp.bfloat16),
            pltpu.VMEM((4, CHUNK_ROWS, SEG), jnp.bfloat16),
            pltpu.VMEM((2, CHUNK_ROWS, SEG), jnp.bfloat16),
            pltpu.VMEM((1, CHUNK_ROWS, SEG), jnp.bfloat16),
            pltpu.VMEM((4, CHUNK_ROWS, SEG), jnp.bfloat16),
            pltpu.VMEM((2, CHUNK_ROWS, SEG), jnp.bfloat16),
            pltpu.VMEM((1, CHUNK_ROWS, SEG), jnp.bfloat16),
            pltpu.VMEM((SQ, DHEADS), jnp.bfloat16),
            pltpu.VMEM((ROWS, DHEADS), jnp.bfloat16),
            pltpu.SemaphoreType.DMA((12,)),
            pltpu.SemaphoreType.DMA((12,)),
        ],
        compiler_params=pltpu.CompilerParams(collective_id=0),
    )(x16, Wq_loc, K_t, V_t, Wo_loc)
    return out_flat.reshape(B, SQ, DMODEL)
